# baseline (device time: 102030 ns/iter reference)
import jax
import jax.numpy as jnp
from jax import lax
from jax.experimental import pallas as pl
from jax.experimental.pallas import tpu as pltpu

N_DEV = 8


def kernel(x, w_mat):
    m, k_per = x.shape
    n = w_mat.shape[1]
    m_out = m // N_DEV

    xb = x.astype(jnp.bfloat16).reshape(N_DEV, m_out, k_per)
    wb = w_mat.astype(jnp.bfloat16)

    def body(x_ref, w_ref, out_ref, stage_ref, recv_ref, send_sems, recv_sems):
        my = lax.axis_index("i")
        left = lax.rem(my + (N_DEV - 1), N_DEV)
        right = lax.rem(my + 1, N_DEV)

        barrier = pltpu.get_barrier_semaphore()
        for nbr in (left, right):
            pl.semaphore_signal(
                barrier, inc=1, device_id=(nbr,),
                device_id_type=pl.DeviceIdType.MESH,
            )
        pl.semaphore_wait(barrier, 2)

        def partial(c):
            return jnp.dot(
                x_ref[c], w_ref[...], preferred_element_type=jnp.float32
            )

        stage_ref[...] = partial(
            lax.rem(my + (N_DEV - 1), N_DEV)
        ).astype(jnp.bfloat16)

        for s in range(N_DEV - 1):
            rdma = pltpu.make_async_remote_copy(
                src_ref=stage_ref,
                dst_ref=recv_ref.at[s],
                send_sem=send_sems.at[s],
                recv_sem=recv_sems.at[s],
                device_id=(right,),
                device_id_type=pl.DeviceIdType.MESH,
            )
            rdma.start()
            p = partial(lax.rem(my + (2 * N_DEV - 2 - s), N_DEV))
            rdma.wait()
            acc = recv_ref[s].astype(jnp.float32) + p
            if s < N_DEV - 2:
                stage_ref[...] = acc.astype(jnp.bfloat16)
            else:
                c = 0.7978845608028654
                out_ref[...] = 0.5 * acc * (
                    1.0 + jnp.tanh(c * (acc + 0.044715 * acc * acc * acc))
                )

    return pl.pallas_call(
        body,
        out_shape=jax.ShapeDtypeStruct((m_out, n), jnp.float32),
        in_specs=[
            pl.BlockSpec(memory_space=pltpu.VMEM),
            pl.BlockSpec(memory_space=pltpu.VMEM),
        ],
        out_specs=pl.BlockSpec(memory_space=pltpu.VMEM),
        scratch_shapes=[
            pltpu.VMEM((m_out, n), jnp.bfloat16),
            pltpu.VMEM((N_DEV - 1, m_out, n), jnp.bfloat16),
            pltpu.SemaphoreType.DMA((N_DEV - 1,)),
            pltpu.SemaphoreType.DMA((N_DEV - 1,)),
        ],
        compiler_params=pltpu.CompilerParams(collective_id=0),
    )(xb, wb)


# device time: 64876 ns/iter; 1.5727x vs baseline; 1.5727x over previous
import jax
import jax.numpy as jnp
from jax import lax
from jax.experimental import pallas as pl
from jax.experimental.pallas import tpu as pltpu

N_DEV = 8


def kernel(x, w_mat):
    m, k_per = x.shape
    n = w_mat.shape[1]
    m_out = m // N_DEV
    nh = n // 2

    xb = x.astype(jnp.bfloat16).reshape(N_DEV, m_out, k_per)
    wb = w_mat.astype(jnp.bfloat16)

    def body(x_ref, w_ref, out_ref,
             stage_r, stage_l, recv_r, recv_l,
             send_sems_r, recv_sems_r, send_sems_l, recv_sems_l):
        my = lax.axis_index("i")
        left = lax.rem(my + (N_DEV - 1), N_DEV)
        right = lax.rem(my + 1, N_DEV)

        barrier = pltpu.get_barrier_semaphore()
        for nbr in (left, right):
            pl.semaphore_signal(
                barrier, inc=1, device_id=(nbr,),
                device_id_type=pl.DeviceIdType.MESH,
            )
        pl.semaphore_wait(barrier, 2)

        def partial_r(c):
            return jnp.dot(
                x_ref[c], w_ref[:, 0:nh], preferred_element_type=jnp.float32
            )

        def partial_l(c):
            return jnp.dot(
                x_ref[c], w_ref[:, nh:n], preferred_element_type=jnp.float32
            )

        stage_r[...] = partial_r(lax.rem(my + (N_DEV - 1), N_DEV)).astype(
            jnp.bfloat16)
        stage_l[...] = partial_l(lax.rem(my + 1, N_DEV)).astype(jnp.bfloat16)

        gelu_c = 0.7978845608028654

        for s in range(N_DEV - 1):
            rdma_r = pltpu.make_async_remote_copy(
                src_ref=stage_r,
                dst_ref=recv_r.at[s],
                send_sem=send_sems_r.at[s],
                recv_sem=recv_sems_r.at[s],
                device_id=(right,),
                device_id_type=pl.DeviceIdType.MESH,
            )
            rdma_l = pltpu.make_async_remote_copy(
                src_ref=stage_l,
                dst_ref=recv_l.at[s],
                send_sem=send_sems_l.at[s],
                recv_sem=recv_sems_l.at[s],
                device_id=(left,),
                device_id_type=pl.DeviceIdType.MESH,
            )
            rdma_r.start()
            rdma_l.start()
            p_r = partial_r(lax.rem(my + (2 * N_DEV - 2 - s), N_DEV))
            p_l = partial_l(lax.rem(my + 2 + s, N_DEV))
            rdma_r.wait()
            acc_r = recv_r[s].astype(jnp.float32) + p_r
            if s < N_DEV - 2:
                stage_r[...] = acc_r.astype(jnp.bfloat16)
            rdma_l.wait()
            acc_l = recv_l[s].astype(jnp.float32) + p_l
            if s < N_DEV - 2:
                stage_l[...] = acc_l.astype(jnp.bfloat16)
            else:
                out_ref[:, 0:nh] = 0.5 * acc_r * (
                    1.0 + jnp.tanh(
                        gelu_c * (acc_r + 0.044715 * acc_r * acc_r * acc_r))
                )
                out_ref[:, nh:n] = 0.5 * acc_l * (
                    1.0 + jnp.tanh(
                        gelu_c * (acc_l + 0.044715 * acc_l * acc_l * acc_l))
                )

    return pl.pallas_call(
        body,
        out_shape=jax.ShapeDtypeStruct((m_out, n), jnp.float32),
        in_specs=[
            pl.BlockSpec(memory_space=pltpu.VMEM),
            pl.BlockSpec(memory_space=pltpu.VMEM),
        ],
        out_specs=pl.BlockSpec(memory_space=pltpu.VMEM),
        scratch_shapes=[
            pltpu.VMEM((m_out, nh), jnp.bfloat16),
            pltpu.VMEM((m_out, nh), jnp.bfloat16),
            pltpu.VMEM((N_DEV - 1, m_out, nh), jnp.bfloat16),
            pltpu.VMEM((N_DEV - 1, m_out, nh), jnp.bfloat16),
            pltpu.SemaphoreType.DMA((N_DEV - 1,)),
            pltpu.SemaphoreType.DMA((N_DEV - 1,)),
            pltpu.SemaphoreType.DMA((N_DEV - 1,)),
            pltpu.SemaphoreType.DMA((N_DEV - 1,)),
        ],
        compiler_params=pltpu.CompilerParams(collective_id=0),
    )(xb, wb)


# device time: 50503 ns/iter; 2.0203x vs baseline; 1.2846x over previous
import jax
import jax.numpy as jnp
from jax import lax
from jax.experimental import pallas as pl
from jax.experimental.pallas import tpu as pltpu

N_DEV = 8
SUB = 2


def kernel(x, w_mat):
    m, k_per = x.shape
    n = w_mat.shape[1]
    m_out = m // N_DEV
    nh = n // 2
    cs = nh // SUB

    xb = x.astype(jnp.bfloat16).reshape(N_DEV, m_out, k_per)
    wb = w_mat.astype(jnp.bfloat16)

    gelu_c = 0.7978845608028654

    def gelu(a):
        return 0.5 * a * (1.0 + jnp.tanh(gelu_c * (a + 0.044715 * a * a * a)))

    def body(x_ref, w_ref, out_ref,
             stage_r, stage_l, recv_r, recv_l,
             send_sems_r, recv_sems_r, send_sems_l, recv_sems_l):
        my = lax.axis_index("i")
        left = lax.rem(my + (N_DEV - 1), N_DEV)
        right = lax.rem(my + 1, N_DEV)

        barrier = pltpu.get_barrier_semaphore()
        for nbr in (left, right):
            pl.semaphore_signal(
                barrier, inc=1, device_id=(nbr,),
                device_id_type=pl.DeviceIdType.MESH,
            )
        pl.semaphore_wait(barrier, 2)

        def partial_r(c):
            return jnp.dot(
                x_ref[c], w_ref[:, 0:nh], preferred_element_type=jnp.float32
            )

        def partial_l(c):
            return jnp.dot(
                x_ref[c], w_ref[:, nh:n], preferred_element_type=jnp.float32
            )

        def make(s, q, stage, recv, ssems, rsems, tgt):
            return pltpu.make_async_remote_copy(
                src_ref=stage.at[:, q * cs:(q + 1) * cs],
                dst_ref=recv.at[s, :, q * cs:(q + 1) * cs],
                send_sem=ssems.at[s, q],
                recv_sem=rsems.at[s, q],
                device_id=(tgt,),
                device_id_type=pl.DeviceIdType.MESH,
            )

        def make_r(s, q):
            return make(s, q, stage_r, recv_r, send_sems_r, recv_sems_r, right)

        def make_l(s, q):
            return make(s, q, stage_l, recv_l, send_sems_l, recv_sems_l, left)

        stage_r[...] = partial_r(lax.rem(my + (N_DEV - 1), N_DEV)).astype(
            jnp.bfloat16)
        stage_l[...] = partial_l(lax.rem(my + 1, N_DEV)).astype(jnp.bfloat16)
        for q in range(SUB):
            make_r(0, q).start()
            make_l(0, q).start()

        for s in range(N_DEV - 1):
            last = s == N_DEV - 2
            p_r = partial_r(lax.rem(my + (2 * N_DEV - 2 - s), N_DEV))
            p_l = partial_l(lax.rem(my + 2 + s, N_DEV))
            for q in range(SUB):
                qs = slice(q * cs, (q + 1) * cs)
                make_r(s, q).wait()
                acc_r = recv_r[s, :, qs].astype(jnp.float32) + p_r[:, qs]
                if not last:
                    stage_r[:, qs] = acc_r.astype(jnp.bfloat16)
                    make_r(s + 1, q).start()
                else:
                    out_ref[:, qs] = gelu(acc_r)
                make_l(s, q).wait()
                acc_l = recv_l[s, :, qs].astype(jnp.float32) + p_l[:, qs]
                if not last:
                    stage_l[:, qs] = acc_l.astype(jnp.bfloat16)
                    make_l(s + 1, q).start()
                else:
                    out_ref[:, nh + q * cs:nh + (q + 1) * cs] = gelu(acc_l)

    return pl.pallas_call(
        body,
        out_shape=jax.ShapeDtypeStruct((m_out, n), jnp.float32),
        in_specs=[
            pl.BlockSpec(memory_space=pltpu.VMEM),
            pl.BlockSpec(memory_space=pltpu.VMEM),
        ],
        out_specs=pl.BlockSpec(memory_space=pltpu.VMEM),
        scratch_shapes=[
            pltpu.VMEM((m_out, nh), jnp.bfloat16),
            pltpu.VMEM((m_out, nh), jnp.bfloat16),
            pltpu.VMEM((N_DEV - 1, m_out, nh), jnp.bfloat16),
            pltpu.VMEM((N_DEV - 1, m_out, nh), jnp.bfloat16),
            pltpu.SemaphoreType.DMA((N_DEV - 1, SUB)),
            pltpu.SemaphoreType.DMA((N_DEV - 1, SUB)),
            pltpu.SemaphoreType.DMA((N_DEV - 1, SUB)),
            pltpu.SemaphoreType.DMA((N_DEV - 1, SUB)),
        ],
        compiler_params=pltpu.CompilerParams(collective_id=0),
    )(xb, wb)
